# baseline (speedup 1.0000x reference)
import functools

import jax
import jax.numpy as jnp
from jax import lax
from jax.experimental import pallas as pl
from jax.experimental.pallas import tpu as pltpu
from jax.experimental.pallas import tpu_sc as plsc

B = 4096
L = 200
WORD_DIM = 50
OUT_DIM = 60
PAD_DIM = 64
TOK = B * L

_info = plsc.get_sparse_core_info()
NC, NS, LANES = _info.num_cores, _info.num_subcores, _info.num_lanes
NW = NC * NS

NSPLIT = 2
TOKS = TOK // NSPLIT
PER_W = TOKS // NW
CHUNK = 128
NBUF = 4
NCHUNK = PER_W // CHUNK
IDX_PER_DMA = 128
NDMA = CHUNK // IDX_PER_DMA


def _sc_embed(word_pad, widx, p1idx, p2idx, p1t, p2t):
    mesh = plsc.VectorSubcoreMesh(core_axis_name="c", subcore_axis_name="s")

    scratch = (
        [pltpu.VMEM((CHUNK,), jnp.int32) for _ in range(NBUF)]
        + [pltpu.VMEM((CHUNK,), jnp.int32) for _ in range(NBUF)]
        + [pltpu.VMEM((CHUNK,), jnp.int32) for _ in range(NBUF)]
        + [pltpu.VMEM((CHUNK, PAD_DIM), jnp.float32)
           for _ in range(NBUF)]
        + [pltpu.VMEM((2 * 2 * L * 5,), jnp.float32)]
        + [pltpu.SemaphoreType.DMA for _ in range(NBUF)]
        + [pltpu.SemaphoreType.DMA for _ in range(NBUF)]
    )

    @functools.partial(
        pl.kernel,
        mesh=mesh,
        out_type=jax.ShapeDtypeStruct((TOKS, PAD_DIM), jnp.float32),
        compiler_params=pltpu.CompilerParams(
            needs_layout_passes=False, use_tc_tiling_on_sc=False),
        scratch_types=scratch,
    )
    def k(word_hbm, widx_hbm, p1idx_hbm, p2idx_hbm, p1t_hbm, p2t_hbm,
          out_hbm, *refs):
        widx_v = refs[0:NBUF]
        p1i_v = refs[NBUF:2 * NBUF]
        p2i_v = refs[2 * NBUF:3 * NBUF]
        rows_v = refs[3 * NBUF:4 * NBUF]
        pcat_v = refs[4 * NBUF]
        gsem = refs[4 * NBUF + 1:5 * NBUF + 1]
        wsem = refs[5 * NBUF + 1:6 * NBUF + 1]

        wid = lax.axis_index("s") * NC + lax.axis_index("c")
        base_w = wid * PER_W
        pltpu.sync_copy(p1t_hbm, pcat_v.at[pl.ds(0, 2 * L * 5)])
        pltpu.sync_copy(p2t_hbm, pcat_v.at[pl.ds(2 * L * 5, 2 * L * 5)])

        def stage_and_fire(ci, p):
            base = base_w + ci * CHUNK
            pltpu.sync_copy(widx_hbm.at[pl.ds(base, CHUNK)], widx_v[p])
            pltpu.sync_copy(p1idx_hbm.at[pl.ds(base, CHUNK)], p1i_v[p])
            pltpu.sync_copy(p2idx_hbm.at[pl.ds(base, CHUNK)], p2i_v[p])
            for di in range(NDMA):
                pltpu.async_copy(
                    word_hbm.at[widx_v[p].at[pl.ds(di * IDX_PER_DMA,
                                                   IDX_PER_DMA)]],
                    rows_v[p].at[pl.ds(di * IDX_PER_DMA, IDX_PER_DMA), :],
                    gsem[p])

        def wait_gather(p):
            pltpu.make_async_copy(
                word_hbm.at[pl.ds(0, CHUNK), :], rows_v[p], gsem[p]).wait()

        def wait_wb(p):
            pltpu.make_async_copy(
                rows_v[p], out_hbm.at[pl.ds(0, CHUNK), :], wsem[p]).wait()

        def pos_fill(p):
            iota = lax.iota(jnp.int32, LANES)
            diags = []
            for s in range(10):
                jj = iota + s
                jj = jnp.where(jj >= 10, jj - 10, jj)
                jj = jnp.where(jj >= 10, jj - 10, jj)
                diags.append(jj)

            def grp(g, carry):
                tb = g * LANES
                t16 = iota + tb
                p1i = p1i_v[p][pl.ds(tb, LANES)] * 5
                p2i = p2i_v[p][pl.ds(tb, LANES)] * 5 + (2 * L * 5 - 5)
                for s in range(10):
                    jj = diags[s]
                    src = jnp.where(jj < 5, p1i + jj, p2i + jj)
                    v = plsc.load_gather(pcat_v, [src])
                    plsc.store_scatter(
                        rows_v[p], [t16, jj + WORD_DIM], v)
                return carry

            lax.fori_loop(0, CHUNK // LANES, grp, 0)

        stage_and_fire(0, 0)

        def body(i4, carry):
            for p in range(NBUF):
                j = NBUF * i4 + p
                q = (p + 1) % NBUF
                wait_gather(p)

                @pl.when(j + 1 < NCHUNK)
                def _():
                    @pl.when(j >= NBUF - 1)
                    def _():
                        wait_wb(q)

                    stage_and_fire(j + 1, q)

                pos_fill(p)
                pltpu.async_copy(
                    rows_v[p],
                    out_hbm.at[pl.ds(base_w + j * CHUNK, CHUNK), :],
                    wsem[p])
            return carry

        lax.fori_loop(0, NCHUNK // NBUF, body, 0)
        for p in range(NBUF):
            wait_wb(p)

    return k(word_pad, widx, p1idx, p2idx, p1t, p2t)


def kernel(word, pos1, pos2, word_table, pos1_table, pos2_table):
    word_pad = jnp.pad(word_table, ((0, 0), (0, PAD_DIM - WORD_DIM)))
    widx = word.reshape(NSPLIT, TOKS)
    p1idx = pos1.reshape(NSPLIT, TOKS)
    p2idx = pos2.reshape(NSPLIT, TOKS)
    p1t = pos1_table.reshape(-1)
    p2t = pos2_table.reshape(-1)
    halves = []
    for s in range(NSPLIT):
        out_pad = _sc_embed(
            word_pad, widx[s], p1idx[s], p2idx[s], p1t, p2t)
        halves.append(
            out_pad[:, :OUT_DIM].reshape(B // NSPLIT, L, OUT_DIM))
    return jnp.concatenate(halves, axis=0)


# speedup vs baseline: 1.0066x; 1.0066x over previous
import functools

import jax
import jax.numpy as jnp
from jax import lax
from jax.experimental import pallas as pl
from jax.experimental.pallas import tpu as pltpu
from jax.experimental.pallas import tpu_sc as plsc

B = 4096
L = 200
WORD_DIM = 50
OUT_DIM = 60
PAD_DIM = 64
TOK = B * L
PTAB = 2 * L * 5

_info = plsc.get_sparse_core_info()
NC, NS, LANES = _info.num_cores, _info.num_subcores, _info.num_lanes
NW = NC * NS

SENT_W = B // NW
SCHUNK = 2
CTOK = SCHUNK * L
NCHUNK = SENT_W // SCHUNK
IDX_PER_DMA = 128


def _sc_embed(word_pad, widx, p1idx, p2idx, p1t, p2t):
    mesh = plsc.VectorSubcoreMesh(core_axis_name="c", subcore_axis_name="s")

    @functools.partial(
        pl.kernel,
        mesh=mesh,
        out_type=jax.ShapeDtypeStruct((B, L, OUT_DIM), jnp.float32),
        compiler_params=pltpu.CompilerParams(
            needs_layout_passes=False, use_tc_tiling_on_sc=False),
        scratch_types=[
            pltpu.VMEM((CTOK,), jnp.int32),
            pltpu.VMEM((CTOK,), jnp.int32),
            pltpu.VMEM((CTOK,), jnp.int32),
            pltpu.VMEM((CTOK,), jnp.int32),
            pltpu.VMEM((CTOK,), jnp.int32),
            pltpu.VMEM((CTOK,), jnp.int32),
            pltpu.VMEM((CTOK, PAD_DIM), jnp.float32),
            pltpu.VMEM((CTOK, PAD_DIM), jnp.float32),
            pltpu.VMEM((SCHUNK, L, OUT_DIM), jnp.float32),
            pltpu.VMEM((SCHUNK, L, OUT_DIM), jnp.float32),
            pltpu.VMEM((2 * PTAB,), jnp.float32),
            pltpu.SemaphoreType.DMA,
            pltpu.SemaphoreType.DMA,
            pltpu.SemaphoreType.DMA,
            pltpu.SemaphoreType.DMA,
        ],
    )
    def k(word_hbm, widx_hbm, p1idx_hbm, p2idx_hbm, p1t_hbm, p2t_hbm,
          out_hbm, widxA, widxB, p1iA, p1iB, p2iA, p2iB,
          rowsA, rowsB, packA, packB, pcat_v,
          gsemA, gsemB, wsemA, wsemB):
        wid = lax.axis_index("s") * NC + lax.axis_index("c")
        sent_w = wid * SENT_W
        base_w = sent_w * L
        pltpu.sync_copy(p1t_hbm, pcat_v.at[pl.ds(0, PTAB)])
        pltpu.sync_copy(p2t_hbm, pcat_v.at[pl.ds(PTAB, PTAB)])

        iota = lax.iota(jnp.int32, LANES)
        wdiag = []
        for s in range(WORD_DIM):
            jj = iota + s
            jj = jnp.where(jj >= WORD_DIM, jj - WORD_DIM, jj)
            wdiag.append(jj)
        pdiag = []
        for s in range(10):
            jj = iota + s
            jj = jnp.where(jj >= 10, jj - 10, jj)
            jj = jnp.where(jj >= 10, jj - 10, jj)
            pdiag.append(jj)

        def stage_and_fire(ci, widx_v, p1i_v, p2i_v, rows, gsem):
            base = base_w + ci * CTOK
            pltpu.sync_copy(widx_hbm.at[pl.ds(base, CTOK)], widx_v)
            pltpu.sync_copy(p1idx_hbm.at[pl.ds(base, CTOK)], p1i_v)
            pltpu.sync_copy(p2idx_hbm.at[pl.ds(base, CTOK)], p2i_v)
            off = 0
            while off < CTOK:
                n = min(IDX_PER_DMA, CTOK - off)
                pltpu.async_copy(
                    word_hbm.at[widx_v.at[pl.ds(off, n)]],
                    rows.at[pl.ds(off, n), :],
                    gsem)
                off += n

        def wait_gather(rows, gsem):
            pltpu.make_async_copy(
                word_hbm.at[pl.ds(0, CTOK), :], rows, gsem).wait()

        def fire_wb(pack, ci, wsem):
            pltpu.async_copy(
                pack,
                out_hbm.at[pl.ds(sent_w + ci * SCHUNK, SCHUNK), :, :],
                wsem)

        def wait_wb(pack, wsem):
            pltpu.make_async_copy(
                pack, out_hbm.at[pl.ds(0, SCHUNK), :, :], wsem).wait()

        def repack(rows, pack, p1i_v, p2i_v):
            def grp(g, carry):
                tb = g * LANES
                t16 = iota + tb
                sent = jnp.where(t16 >= L, 1, 0)
                l16 = t16 - L * sent
                p1i = p1i_v[pl.ds(tb, LANES)] * 5
                p2i = p2i_v[pl.ds(tb, LANES)] * 5 + (PTAB - 5)
                for s in range(WORD_DIM):
                    jj = wdiag[s]
                    v = plsc.load_gather(rows, [t16, jj])
                    plsc.store_scatter(pack, [sent, l16, jj], v)
                for s in range(10):
                    jj = pdiag[s]
                    src = jnp.where(jj < 5, p1i + jj, p2i + jj)
                    v = plsc.load_gather(pcat_v, [src])
                    plsc.store_scatter(pack, [sent, l16, jj + WORD_DIM], v)
                return carry

            lax.fori_loop(0, CTOK // LANES, grp, 0)

        stage_and_fire(0, widxA, p1iA, p2iA, rowsA, gsemA)

        def body(i2, carry):
            jA = 2 * i2
            jB = jA + 1
            wait_gather(rowsA, gsemA)
            stage_and_fire(jB, widxB, p1iB, p2iB, rowsB, gsemB)

            @pl.when(jA >= 2)
            def _():
                wait_wb(packA, wsemA)

            repack(rowsA, packA, p1iA, p2iA)
            fire_wb(packA, jA, wsemA)

            wait_gather(rowsB, gsemB)

            @pl.when(jB + 1 < NCHUNK)
            def _():
                stage_and_fire(jB + 1, widxA, p1iA, p2iA, rowsA, gsemA)

            @pl.when(jB >= 2)
            def _():
                wait_wb(packB, wsemB)

            repack(rowsB, packB, p1iB, p2iB)
            fire_wb(packB, jB, wsemB)
            return carry

        lax.fori_loop(0, NCHUNK // 2, body, 0)
        wait_wb(packA, wsemA)
        wait_wb(packB, wsemB)

    return k(word_pad, widx, p1idx, p2idx, p1t, p2t)


def kernel(word, pos1, pos2, word_table, pos1_table, pos2_table):
    word_pad = jnp.pad(word_table, ((0, 0), (0, PAD_DIM - WORD_DIM)))
    return _sc_embed(
        word_pad,
        word.reshape(-1),
        pos1.reshape(-1),
        pos2.reshape(-1),
        pos1_table.reshape(-1),
        pos2_table.reshape(-1),
    )


# speedup vs baseline: 1.1884x; 1.1805x over previous
import functools

import jax
import jax.numpy as jnp
from jax import lax
from jax.experimental import pallas as pl
from jax.experimental.pallas import tpu as pltpu
from jax.experimental.pallas import tpu_sc as plsc

B = 4096
L = 200
WORD_DIM = 50
OUT_DIM = 60
PAD_DIM = 64
TOK = B * L

_info = plsc.get_sparse_core_info()
NC, NS, LANES = _info.num_cores, _info.num_subcores, _info.num_lanes
NW = NC * NS

NSPLIT = 1
TOKS = TOK // NSPLIT
PER_W = TOKS // NW
CHUNK = 256
NBUF = 4
NCHUNK = PER_W // CHUNK
IDX_PER_DMA = 128
NDMA = CHUNK // IDX_PER_DMA


def _sc_embed(word_pad, widx, p1idx, p2idx, p1t, p2t):
    mesh = plsc.VectorSubcoreMesh(core_axis_name="c", subcore_axis_name="s")

    scratch = (
        [pltpu.VMEM((CHUNK,), jnp.int32) for _ in range(NBUF)]
        + [pltpu.VMEM((CHUNK,), jnp.int32) for _ in range(NBUF)]
        + [pltpu.VMEM((CHUNK,), jnp.int32) for _ in range(NBUF)]
        + [pltpu.VMEM((CHUNK, PAD_DIM), jnp.float32)
           for _ in range(NBUF)]
        + [pltpu.VMEM((2 * 2 * L * 5,), jnp.float32)]
        + [pltpu.SemaphoreType.DMA for _ in range(NBUF)]
        + [pltpu.SemaphoreType.DMA for _ in range(NBUF)]
    )

    @functools.partial(
        pl.kernel,
        mesh=mesh,
        out_type=jax.ShapeDtypeStruct((TOKS, PAD_DIM), jnp.float32),
        compiler_params=pltpu.CompilerParams(
            needs_layout_passes=False, use_tc_tiling_on_sc=False),
        scratch_types=scratch,
    )
    def k(word_hbm, widx_hbm, p1idx_hbm, p2idx_hbm, p1t_hbm, p2t_hbm,
          out_hbm, *refs):
        widx_v = refs[0:NBUF]
        p1i_v = refs[NBUF:2 * NBUF]
        p2i_v = refs[2 * NBUF:3 * NBUF]
        rows_v = refs[3 * NBUF:4 * NBUF]
        pcat_v = refs[4 * NBUF]
        gsem = refs[4 * NBUF + 1:5 * NBUF + 1]
        wsem = refs[5 * NBUF + 1:6 * NBUF + 1]

        wid = lax.axis_index("s") * NC + lax.axis_index("c")
        base_w = wid * PER_W
        pltpu.sync_copy(p1t_hbm, pcat_v.at[pl.ds(0, 2 * L * 5)])
        pltpu.sync_copy(p2t_hbm, pcat_v.at[pl.ds(2 * L * 5, 2 * L * 5)])

        def stage_and_fire(ci, p):
            base = base_w + ci * CHUNK
            pltpu.sync_copy(widx_hbm.at[pl.ds(base, CHUNK)], widx_v[p])
            pltpu.sync_copy(p1idx_hbm.at[pl.ds(base, CHUNK)], p1i_v[p])
            pltpu.sync_copy(p2idx_hbm.at[pl.ds(base, CHUNK)], p2i_v[p])
            for di in range(NDMA):
                pltpu.async_copy(
                    word_hbm.at[widx_v[p].at[pl.ds(di * IDX_PER_DMA,
                                                   IDX_PER_DMA)]],
                    rows_v[p].at[pl.ds(di * IDX_PER_DMA, IDX_PER_DMA), :],
                    gsem[p])

        def wait_gather(p):
            pltpu.make_async_copy(
                word_hbm.at[pl.ds(0, CHUNK), :], rows_v[p], gsem[p]).wait()

        def wait_wb(p):
            pltpu.make_async_copy(
                rows_v[p], out_hbm.at[pl.ds(0, CHUNK), :], wsem[p]).wait()

        def pos_fill(p):
            iota = lax.iota(jnp.int32, LANES)
            diags = []
            for s in range(10):
                jj = iota + s
                jj = jnp.where(jj >= 10, jj - 10, jj)
                jj = jnp.where(jj >= 10, jj - 10, jj)
                diags.append(jj)

            def grp(g, carry):
                tb = g * LANES
                t16 = iota + tb
                p1i = p1i_v[p][pl.ds(tb, LANES)] * 5
                p2i = p2i_v[p][pl.ds(tb, LANES)] * 5 + (2 * L * 5 - 5)
                for s in range(10):
                    jj = diags[s]
                    src = jnp.where(jj < 5, p1i + jj, p2i + jj)
                    v = plsc.load_gather(pcat_v, [src])
                    plsc.store_scatter(
                        rows_v[p], [t16, jj + WORD_DIM], v)
                return carry

            lax.fori_loop(0, CHUNK // LANES, grp, 0)

        stage_and_fire(0, 0)

        def body(i4, carry):
            for p in range(NBUF):
                j = NBUF * i4 + p
                q = (p + 1) % NBUF
                wait_gather(p)

                @pl.when(j + 1 < NCHUNK)
                def _():
                    @pl.when(j >= NBUF - 1)
                    def _():
                        wait_wb(q)

                    stage_and_fire(j + 1, q)

                pos_fill(p)
                pltpu.async_copy(
                    rows_v[p],
                    out_hbm.at[pl.ds(base_w + j * CHUNK, CHUNK), :],
                    wsem[p])
            return carry

        lax.fori_loop(0, NCHUNK // NBUF, body, 0)
        for p in range(NBUF):
            wait_wb(p)

    return k(word_pad, widx, p1idx, p2idx, p1t, p2t)


def kernel(word, pos1, pos2, word_table, pos1_table, pos2_table):
    word_pad = jnp.pad(word_table, ((0, 0), (0, PAD_DIM - WORD_DIM)))
    out_pad = _sc_embed(
        word_pad,
        word.reshape(-1),
        pos1.reshape(-1),
        pos2.reshape(-1),
        pos1_table.reshape(-1),
        pos2_table.reshape(-1),
    )
    return out_pad[:, :OUT_DIM].reshape(B, L, OUT_DIM)
